# baseline (speedup 1.0000x reference)
import jax
import jax.numpy as jnp
from jax.experimental import pallas as pl
from jax.experimental.pallas import tpu as pltpu

_B, _N = 256, 512
_EMB, _KPT = 128, 17
_TOKEN_DIM = 128
_G = 8
_R = _G * _N
_S = 4
_C = _R // _S


def _emb_copy(emb_hbm, emb_buf, sems, step, buf, s):
    return pltpu.make_async_copy(
        emb_hbm.at[pl.ds(step * _R + s * _C, _C), :],
        emb_buf.at[buf, pl.ds(s * _C, _C), :],
        sems.at[buf, s])


def _out_copy(out_hbm, out_buf, sems, step, buf, s):
    return pltpu.make_async_copy(
        out_buf.at[buf, pl.ds(s * _C, _C), :],
        out_hbm.at[pl.ds(step * _R + s * _C, _C), :],
        sems.at[buf, s])


def _body(mask_ref, vis_ref, bbox_ref, kpt_ref,
          wembT_ref, wvis_ref, wbboxT_ref, wkptT_ref, ab_ref, sb_ref,
          emb_hbm, out_hbm,
          emb_buf, out_buf, load_sem, store_sem):
    f32 = jnp.float32
    i = pl.program_id(0)
    nsteps = pl.num_programs(0)
    cur = jax.lax.rem(i, 2)
    nxt = jax.lax.rem(i + 1, 2)

    @pl.when(i == 0)
    def _():
        for s in range(_S):
            _emb_copy(emb_hbm, emb_buf, load_sem, i, cur, s).start()

    @pl.when(i + 1 < nsteps)
    def _():
        for s in range(_S):
            _emb_copy(emb_hbm, emb_buf, load_sem, i + 1, nxt, s).start()

    for s in range(_S):
        _emb_copy(emb_hbm, emb_buf, load_sem, i, cur, s).wait()

    @pl.when(i >= 2)
    def _():
        for s in range(_S):
            _out_copy(out_hbm, out_buf, store_sem, i - 2, cur, s).wait()

    dn_t = (((0,), (0,)), ((), ()))
    ws = jnp.concatenate([wbboxT_ref[...], wvis_ref[...],
                          ab_ref[...], sb_ref[...]], axis=0)
    ones = jnp.ones((1, _N), f32)
    mask_cols = jnp.transpose(mask_ref[:, 0, :], (1, 0))
    for j in range(_G):
        acc = jax.lax.dot_general(emb_buf[cur, pl.ds(j * _N, _N), :],
                                  wembT_ref[...],
                                  (((1,), (0,)), ((), ())),
                                  preferred_element_type=f32)
        acc += jax.lax.dot_general(kpt_ref[:, j, 0, :], wkptT_ref[...], dn_t,
                                   preferred_element_type=f32)
        xs = jnp.concatenate([bbox_ref[j], vis_ref[j], ones, ones], axis=0)
        acc += jax.lax.dot_general(xs, ws, dn_t, preferred_element_type=f32)
        out_buf[cur, pl.ds(j * _N, _N), :] = acc * mask_cols[:, j:j + 1]
        if (j + 1) % (_G // _S) == 0:
            _out_copy(out_hbm, out_buf, store_sem, i, cur,
                      (j + 1) // (_G // _S) - 1).start()

    @pl.when(i == nsteps - 1)
    def _():
        for s in range(_S):
            _out_copy(out_hbm, out_buf, store_sem, i - 1, nxt, s).wait()
            _out_copy(out_hbm, out_buf, store_sem, i, cur, s).wait()


def kernel(feats_masks, embeddings, visibility_scores, bbox_ltwh,
           keypoints_xyc, app_W, app_b, st_W, st_b):
    m = _B * _N
    maskf = feats_masks.astype(jnp.float32).reshape(_B, 1, _N)
    emb = embeddings.reshape(m, _EMB)
    vis = visibility_scores.reshape(_B, 1, _N)
    bboxT = bbox_ltwh.transpose(0, 2, 1)
    kptT = keypoints_xyc.transpose(2, 3, 0, 1).reshape(_KPT * 3, _B, 1, _N)
    app_WT = app_W.T
    wembT = app_WT[:_EMB]
    wvis = app_WT[_EMB:]
    st_WT = st_W.T
    wbboxT = st_WT[:4]
    wkptT = st_WT[4:]
    ab = app_b.reshape(1, _TOKEN_DIM)
    sb = st_b.reshape(1, _TOKEN_DIM)

    grid = (_B // _G,)
    rep = lambda i: (0, 0)
    out = pl.pallas_call(
        _body,
        grid=grid,
        in_specs=[
            pl.BlockSpec((_G, 1, _N), lambda i: (i, 0, 0)),
            pl.BlockSpec((_G, 1, _N), lambda i: (i, 0, 0)),
            pl.BlockSpec((_G, 4, _N), lambda i: (i, 0, 0)),
            pl.BlockSpec((_KPT * 3, _G, 1, _N), lambda i: (0, i, 0, 0)),
            pl.BlockSpec((_EMB, _TOKEN_DIM), rep),
            pl.BlockSpec((1, _TOKEN_DIM), rep),
            pl.BlockSpec((4, _TOKEN_DIM), rep),
            pl.BlockSpec((_KPT * 3, _TOKEN_DIM), rep),
            pl.BlockSpec((1, _TOKEN_DIM), rep),
            pl.BlockSpec((1, _TOKEN_DIM), rep),
            pl.BlockSpec(memory_space=pl.ANY),
        ],
        out_specs=pl.BlockSpec(memory_space=pl.ANY),
        out_shape=jax.ShapeDtypeStruct((m, _TOKEN_DIM), jnp.float32),
        scratch_shapes=[
            pltpu.VMEM((2, _R, _EMB), jnp.float32),
            pltpu.VMEM((2, _R, _TOKEN_DIM), jnp.float32),
            pltpu.SemaphoreType.DMA((2, _S)),
            pltpu.SemaphoreType.DMA((2, _S)),
        ],
        compiler_params=pltpu.CompilerParams(
            dimension_semantics=("arbitrary",),
        ),
    )(maskf, vis, bboxT, kptT, wembT, wvis, wbboxT, wkptT, ab, sb, emb)
    return out.reshape(_B, _N, _TOKEN_DIM)


# speedup vs baseline: 1.0771x; 1.0771x over previous
import jax
import jax.numpy as jnp
from jax.experimental import pallas as pl
from jax.experimental.pallas import tpu as pltpu

_B, _N = 256, 512
_EMB, _KPT = 128, 17
_TOKEN_DIM = 128
_G = 16
_R = _G * _N
_S = 4
_C = _R // _S


def _emb_copy(emb_hbm, emb_buf, sems, step, buf, s):
    return pltpu.make_async_copy(
        emb_hbm.at[pl.ds(step * _R + s * _C, _C), :],
        emb_buf.at[buf, pl.ds(s * _C, _C), :],
        sems.at[buf, s])


def _out_copy(out_hbm, out_buf, sems, step, buf, s):
    return pltpu.make_async_copy(
        out_buf.at[buf, pl.ds(s * _C, _C), :],
        out_hbm.at[pl.ds(step * _R + s * _C, _C), :],
        sems.at[buf, s])


def _body(mask_ref, vis_ref, bbox_ref, kpt_ref,
          wembT_ref, wvis_ref, wbboxT_ref, wkptT_ref, ab_ref, sb_ref,
          emb_hbm, out_hbm,
          emb_buf, out_buf, load_sem, store_sem):
    f32 = jnp.float32
    i = pl.program_id(0)
    nsteps = pl.num_programs(0)
    cur = jax.lax.rem(i, 2)
    nxt = jax.lax.rem(i + 1, 2)

    @pl.when(i == 0)
    def _():
        for s in range(_S):
            _emb_copy(emb_hbm, emb_buf, load_sem, i, cur, s).start()

    @pl.when(i + 1 < nsteps)
    def _():
        for s in range(_S):
            _emb_copy(emb_hbm, emb_buf, load_sem, i + 1, nxt, s).start()

    for s in range(_S):
        _emb_copy(emb_hbm, emb_buf, load_sem, i, cur, s).wait()

    @pl.when(i >= 2)
    def _():
        for s in range(_S):
            _out_copy(out_hbm, out_buf, store_sem, i - 2, cur, s).wait()

    bf16 = jnp.bfloat16
    dn_t = (((0,), (0,)), ((), ()))
    ws = jnp.concatenate([wbboxT_ref[...], wvis_ref[...],
                          ab_ref[...], sb_ref[...]], axis=0).astype(bf16)
    wemb16 = wembT_ref[...].astype(bf16)
    wkpt16 = wkptT_ref[...].astype(bf16)
    ones = jnp.ones((1, _N), bf16)
    mask_cols = jnp.transpose(mask_ref[:, 0, :], (1, 0))
    for j in range(_G):
        acc = jax.lax.dot_general(emb_buf[cur, pl.ds(j * _N, _N), :].astype(bf16),
                                  wemb16,
                                  (((1,), (0,)), ((), ())),
                                  preferred_element_type=f32)
        acc += jax.lax.dot_general(kpt_ref[:, j, 0, :].astype(bf16), wkpt16,
                                   dn_t, preferred_element_type=f32)
        xs = jnp.concatenate([bbox_ref[j].astype(bf16), vis_ref[j].astype(bf16),
                              ones, ones], axis=0)
        acc += jax.lax.dot_general(xs, ws, dn_t, preferred_element_type=f32)
        out_buf[cur, pl.ds(j * _N, _N), :] = acc * mask_cols[:, j:j + 1]
        if (j + 1) % (_G // _S) == 0:
            _out_copy(out_hbm, out_buf, store_sem, i, cur,
                      (j + 1) // (_G // _S) - 1).start()

    @pl.when(i == nsteps - 1)
    def _():
        for s in range(_S):
            _out_copy(out_hbm, out_buf, store_sem, i - 1, nxt, s).wait()
            _out_copy(out_hbm, out_buf, store_sem, i, cur, s).wait()


def kernel(feats_masks, embeddings, visibility_scores, bbox_ltwh,
           keypoints_xyc, app_W, app_b, st_W, st_b):
    m = _B * _N
    maskf = feats_masks.astype(jnp.float32).reshape(_B, 1, _N)
    emb = embeddings.reshape(m, _EMB)
    vis = visibility_scores.reshape(_B, 1, _N)
    bboxT = bbox_ltwh.transpose(0, 2, 1)
    kptT = keypoints_xyc.transpose(2, 3, 0, 1).reshape(_KPT * 3, _B, 1, _N)
    app_WT = app_W.T
    wembT = app_WT[:_EMB]
    wvis = app_WT[_EMB:]
    st_WT = st_W.T
    wbboxT = st_WT[:4]
    wkptT = st_WT[4:]
    ab = app_b.reshape(1, _TOKEN_DIM)
    sb = st_b.reshape(1, _TOKEN_DIM)

    grid = (_B // _G,)
    rep = lambda i: (0, 0)
    out = pl.pallas_call(
        _body,
        grid=grid,
        in_specs=[
            pl.BlockSpec((_G, 1, _N), lambda i: (i, 0, 0)),
            pl.BlockSpec((_G, 1, _N), lambda i: (i, 0, 0)),
            pl.BlockSpec((_G, 4, _N), lambda i: (i, 0, 0)),
            pl.BlockSpec((_KPT * 3, _G, 1, _N), lambda i: (0, i, 0, 0)),
            pl.BlockSpec((_EMB, _TOKEN_DIM), rep),
            pl.BlockSpec((1, _TOKEN_DIM), rep),
            pl.BlockSpec((4, _TOKEN_DIM), rep),
            pl.BlockSpec((_KPT * 3, _TOKEN_DIM), rep),
            pl.BlockSpec((1, _TOKEN_DIM), rep),
            pl.BlockSpec((1, _TOKEN_DIM), rep),
            pl.BlockSpec(memory_space=pl.ANY),
        ],
        out_specs=pl.BlockSpec(memory_space=pl.ANY),
        out_shape=jax.ShapeDtypeStruct((m, _TOKEN_DIM), jnp.float32),
        scratch_shapes=[
            pltpu.VMEM((2, _R, _EMB), jnp.float32),
            pltpu.VMEM((2, _R, _TOKEN_DIM), jnp.float32),
            pltpu.SemaphoreType.DMA((2, _S)),
            pltpu.SemaphoreType.DMA((2, _S)),
        ],
        compiler_params=pltpu.CompilerParams(
            dimension_semantics=("arbitrary",),
        ),
    )(maskf, vis, bboxT, kptT, wembT, wvis, wbboxT, wkptT, ab, sb, emb)
    return out.reshape(_B, _N, _TOKEN_DIM)
